# baseline (speedup 1.0000x reference)
import functools

import jax
import jax.numpy as jnp
from jax import lax
from jax.experimental import pallas as pl
from jax.experimental.pallas import tpu as pltpu
from jax.experimental.pallas import tpu_sc as plsc

VOCAB = 1000000
LENGTH = 200
DIM = 64
BATCH = 4096


def _positional_encoding(length, dim, n=10000):
    half_dim = dim // 2
    pos = jnp.arange(length, dtype=jnp.float32).reshape(-1, 1)
    i = jnp.arange(half_dim, dtype=jnp.float32).reshape(1, -1)
    denom = jnp.power(jnp.float32(n), -i / half_dim)
    args = pos * denom
    sin = jnp.expand_dims(jnp.sin(args), axis=-1)
    cos = jnp.expand_dims(jnp.cos(args), axis=-1)
    return jnp.concatenate([sin, cos], axis=-1).reshape(length, dim)


def _make_sc_kernel(num_cores, num_subcores):
    nw = num_cores * num_subcores
    rows_per_w = BATCH // nw
    mesh = plsc.VectorSubcoreMesh(core_axis_name="c", subcore_axis_name="s")

    @functools.partial(
        pl.kernel,
        mesh=mesh,
        out_type=jax.ShapeDtypeStruct((BATCH, LENGTH, DIM), jnp.float32),
        scratch_types=[
            pltpu.VMEM((rows_per_w, LENGTH), jnp.int32),
            pltpu.VMEM((LENGTH, DIM), jnp.float32),
            pltpu.VMEM((LENGTH, DIM), jnp.float32),
            pltpu.SemaphoreType.DMA,
        ],
        compiler_params=pltpu.CompilerParams(use_tc_tiling_on_sc=False),
    )
    def sc_kernel(inputs_hbm, table_hbm, pe_hbm, out_hbm, idx_v, pe_v, rows_v, gsem):
        wid = lax.axis_index("s") * num_cores + lax.axis_index("c")
        base = wid * rows_per_w
        pltpu.sync_copy(inputs_hbm.at[pl.ds(base, rows_per_w)], idx_v)
        pltpu.sync_copy(pe_hbm, pe_v)

        @pl.loop(0, rows_per_w)
        def _row(j):
            c0 = pltpu.async_copy(
                table_hbm.at[idx_v.at[j, pl.ds(0, 128)]],
                rows_v.at[pl.ds(0, 128)],
                gsem,
            )
            c1 = pltpu.async_copy(
                table_hbm.at[idx_v.at[j, pl.ds(128, LENGTH - 128)]],
                rows_v.at[pl.ds(128, LENGTH - 128)],
                gsem,
            )
            c0.wait()
            c1.wait()

            @pl.loop(0, LENGTH)
            def _add(r):
                for q in range(DIM // 16):
                    sl = pl.ds(q * 16, 16)
                    rows_v[r, sl] = rows_v[r, sl] + pe_v[r, sl]

            pltpu.sync_copy(rows_v, out_hbm.at[base + j])

    return sc_kernel


def kernel(inputs, table):
    pe = _positional_encoding(LENGTH, DIM)
    info = plsc.get_sparse_core_info()
    sc_kernel = _make_sc_kernel(info.num_cores, info.num_subcores)
    return sc_kernel(inputs.astype(jnp.int32), table, pe)
